# baseline (speedup 1.0000x reference)
import functools

import jax
import jax.numpy as jnp
from jax.experimental import pallas as pl
from jax.experimental.pallas import tpu as pltpu

_BH = 128
_CH = 4


def _grad_kernel(x_ref, halo_ref, out_ref, *, h_blks, ch):
    i = pl.program_id(0)
    cb = pl.program_id(1)
    bh, w = x_ref.shape[2], x_ref.shape[3]

    first = (i % h_blks) == 0
    fscale = jnp.where(first, 0.0, 1.0)
    lanes_row = jax.lax.broadcasted_iota(jnp.int32, (1, w), 1)
    wmask = jnp.where(lanes_row == 0, 0.0, 1.0)

    acc = None
    acc0 = None
    for ci in range(ch):
        xb = x_ref[0, ci]
        lw = pltpu.roll(xb, 1, axis=1) * wmask
        up = pltpu.roll(xb, 1, axis=0)
        dw = (lw - xb).astype(jnp.bfloat16)
        dh = (up - xb).astype(jnp.bfloat16)
        s = dw * dw + dh * dh
        f = s * jax.lax.rsqrt(s + 1e-30)
        acc = f if acc is None else acc + f
        x0 = x_ref[0, ci, 0:1, :]
        xl = x_ref[0, ci, bh - 1:bh, :]
        lw0 = pltpu.roll(x0, 1, axis=1) * wmask
        dw0 = lw0 - x0
        dh0 = halo_ref[0, ci, 7:8, :] * fscale - x0
        dh0w = xl - x0
        sq0 = dw0 * dw0
        s0 = sq0 + dh0 * dh0
        s0w = sq0 + dh0w * dh0w
        f0 = s0 * jax.lax.rsqrt(s0 + 1e-30)
        f0w = s0w * jax.lax.rsqrt(s0w + 1e-30)
        d0 = f0 - f0w
        acc0 = d0 if acc0 is None else acc0 + d0

    @pl.when(cb == 0)
    def _():
        out_ref[0] = acc

    @pl.when(cb > 0)
    def _():
        out_ref[0] = out_ref[0] + acc

    out_ref[0, 0:1, :] = out_ref[0, 0:1, :] + acc0.astype(jnp.bfloat16)


def _box_kernel(g_ref, a_ref, out_ref):
    ab = a_ref[...]
    g16 = g_ref[0]
    t = jnp.dot(ab, g16, preferred_element_type=jnp.float32)
    out_ref[0] = jnp.dot(t.astype(jnp.bfloat16), ab,
                         preferred_element_type=jnp.float32)


def kernel(input) -> jnp.ndarray:
    x = input
    b, nc, hdim, wdim = x.shape
    r = wdim // 40
    bh = _BH
    ch = _CH
    h_blks = hdim // bh

    g = pl.pallas_call(
        functools.partial(_grad_kernel, h_blks=h_blks, ch=ch),
        grid=(b * h_blks, nc // ch),
        in_specs=[
            pl.BlockSpec(
                (1, ch, bh, wdim),
                lambda i, cb: (i // h_blks, cb, i % h_blks, 0),
            ),
            pl.BlockSpec(
                (1, ch, 8, wdim),
                lambda i, cb: (
                    i // h_blks,
                    cb,
                    jnp.maximum((i % h_blks) * (bh // 8) - 1, 0),
                    0,
                ),
            ),
        ],
        out_specs=pl.BlockSpec((1, bh, wdim), lambda i, cb: (i // h_blks, i % h_blks, 0)),
        out_shape=jax.ShapeDtypeStruct((b, hdim, wdim), jnp.bfloat16),
        compiler_params=pltpu.CompilerParams(
            dimension_semantics=("parallel", "arbitrary"),
            vmem_limit_bytes=48 * 1024 * 1024,
        ),
    )(x, x)

    idx = jnp.arange(hdim)
    band = (jnp.abs(idx[:, None] - idx[None, :]) <= r).astype(jnp.bfloat16)

    out = pl.pallas_call(
        _box_kernel,
        grid=(b,),
        in_specs=[
            pl.BlockSpec((1, hdim, wdim), lambda bi: (bi, 0, 0)),
            pl.BlockSpec((hdim, hdim), lambda bi: (0, 0)),
        ],
        out_specs=pl.BlockSpec((1, hdim, wdim), lambda bi: (bi, 0, 0)),
        out_shape=jax.ShapeDtypeStruct((b, hdim, wdim), jnp.float32),
        compiler_params=pltpu.CompilerParams(
            dimension_semantics=("parallel",),
            vmem_limit_bytes=48 * 1024 * 1024,
        ),
    )(g, band)
    return out


# speedup vs baseline: 1.3998x; 1.3998x over previous
import functools

import jax
import jax.numpy as jnp
from jax.experimental import pallas as pl
from jax.experimental.pallas import tpu as pltpu

_BH = 128
_CH = 16


def _grad_kernel(x_ref, halo_ref, out_ref, *, h_blks, ch):
    i = pl.program_id(0)
    cb = pl.program_id(1)
    bh, w = x_ref.shape[2], x_ref.shape[3]

    first = (i % h_blks) == 0
    fscale = jnp.where(first, 0.0, 1.0)
    lanes_row = jax.lax.broadcasted_iota(jnp.int32, (1, w), 1)
    wmask = jnp.where(lanes_row == 0, 0.0, 1.0)

    acc = None
    acc0 = None
    for ci in range(ch):
        xb = x_ref[0, ci]
        lw = pltpu.roll(xb, 1, axis=1) * wmask
        up = pltpu.roll(xb, 1, axis=0)
        dw = (lw - xb).astype(jnp.bfloat16)
        dh = (up - xb).astype(jnp.bfloat16)
        s = dw * dw + dh * dh
        f = s * jax.lax.rsqrt(s + 1e-30)
        acc = f if acc is None else acc + f
        x0 = x_ref[0, ci, 0:1, :]
        xl = x_ref[0, ci, bh - 1:bh, :]
        lw0 = pltpu.roll(x0, 1, axis=1) * wmask
        dw0 = lw0 - x0
        dh0 = halo_ref[0, ci, 7:8, :] * fscale - x0
        dh0w = xl - x0
        sq0 = dw0 * dw0
        s0 = sq0 + dh0 * dh0
        s0w = sq0 + dh0w * dh0w
        f0 = s0 * jax.lax.rsqrt(s0 + 1e-30)
        f0w = s0w * jax.lax.rsqrt(s0w + 1e-30)
        d0 = f0 - f0w
        acc0 = d0 if acc0 is None else acc0 + d0

    @pl.when(cb == 0)
    def _():
        out_ref[0] = acc

    @pl.when(cb > 0)
    def _():
        out_ref[0] = out_ref[0] + acc

    out_ref[0, 0:1, :] = out_ref[0, 0:1, :] + acc0.astype(jnp.bfloat16)


def _box_kernel(g_ref, a_ref, out_ref):
    ab = a_ref[...]
    g16 = g_ref[0]
    t = jnp.dot(ab, g16, preferred_element_type=jnp.float32)
    out_ref[0] = jnp.dot(t.astype(jnp.bfloat16), ab,
                         preferred_element_type=jnp.float32)


def kernel(input) -> jnp.ndarray:
    x = input
    b, nc, hdim, wdim = x.shape
    r = wdim // 40
    bh = _BH
    ch = _CH
    h_blks = hdim // bh

    g = pl.pallas_call(
        functools.partial(_grad_kernel, h_blks=h_blks, ch=ch),
        grid=(b * h_blks, nc // ch),
        in_specs=[
            pl.BlockSpec(
                (1, ch, bh, wdim),
                lambda i, cb: (i // h_blks, cb, i % h_blks, 0),
            ),
            pl.BlockSpec(
                (1, ch, 8, wdim),
                lambda i, cb: (
                    i // h_blks,
                    cb,
                    jnp.maximum((i % h_blks) * (bh // 8) - 1, 0),
                    0,
                ),
            ),
        ],
        out_specs=pl.BlockSpec((1, bh, wdim), lambda i, cb: (i // h_blks, i % h_blks, 0)),
        out_shape=jax.ShapeDtypeStruct((b, hdim, wdim), jnp.bfloat16),
        compiler_params=pltpu.CompilerParams(
            dimension_semantics=("parallel", "arbitrary"),
            vmem_limit_bytes=48 * 1024 * 1024,
        ),
    )(x, x)

    idx = jnp.arange(hdim)
    band = (jnp.abs(idx[:, None] - idx[None, :]) <= r).astype(jnp.bfloat16)

    out = pl.pallas_call(
        _box_kernel,
        grid=(b,),
        in_specs=[
            pl.BlockSpec((1, hdim, wdim), lambda bi: (bi, 0, 0)),
            pl.BlockSpec((hdim, hdim), lambda bi: (0, 0)),
        ],
        out_specs=pl.BlockSpec((1, hdim, wdim), lambda bi: (bi, 0, 0)),
        out_shape=jax.ShapeDtypeStruct((b, hdim, wdim), jnp.float32),
        compiler_params=pltpu.CompilerParams(
            dimension_semantics=("parallel",),
            vmem_limit_bytes=48 * 1024 * 1024,
        ),
    )(g, band)
    return out


# speedup vs baseline: 1.4745x; 1.0534x over previous
import functools

import jax
import jax.numpy as jnp
from jax.experimental import pallas as pl
from jax.experimental.pallas import tpu as pltpu

_BH = 128
_CH = 32


def _grad_kernel(x_ref, halo_ref, out_ref, *, h_blks, ch):
    i = pl.program_id(0)
    cb = pl.program_id(1)
    bh, w = x_ref.shape[2], x_ref.shape[3]

    first = (i % h_blks) == 0
    fscale = jnp.where(first, 0.0, 1.0)
    lanes_row = jax.lax.broadcasted_iota(jnp.int32, (1, w), 1)
    wmask = jnp.where(lanes_row == 0, 0.0, 1.0)

    acc = None
    acc0 = None
    for ci in range(ch):
        xb = x_ref[0, ci]
        lw = pltpu.roll(xb, 1, axis=1) * wmask
        up = pltpu.roll(xb, 1, axis=0)
        dw = (lw - xb).astype(jnp.bfloat16)
        dh = (up - xb).astype(jnp.bfloat16)
        s = dw * dw + dh * dh
        f = s * jax.lax.rsqrt(s + 1e-30)
        acc = f if acc is None else acc + f
        x0 = x_ref[0, ci, 0:1, :]
        xl = x_ref[0, ci, bh - 1:bh, :]
        lw0 = pltpu.roll(x0, 1, axis=1) * wmask
        dw0 = lw0 - x0
        dh0 = halo_ref[0, ci, 7:8, :] * fscale - x0
        dh0w = xl - x0
        sq0 = dw0 * dw0
        s0 = sq0 + dh0 * dh0
        s0w = sq0 + dh0w * dh0w
        f0 = s0 * jax.lax.rsqrt(s0 + 1e-30)
        f0w = s0w * jax.lax.rsqrt(s0w + 1e-30)
        d0 = f0 - f0w
        acc0 = d0 if acc0 is None else acc0 + d0

    @pl.when(cb == 0)
    def _():
        out_ref[0] = acc

    @pl.when(cb > 0)
    def _():
        out_ref[0] = out_ref[0] + acc

    out_ref[0, 0:1, :] = out_ref[0, 0:1, :] + acc0.astype(jnp.bfloat16)


def _box_kernel(g_ref, a_ref, out_ref):
    ab = a_ref[...]
    g16 = g_ref[0]
    t = jnp.dot(ab, g16, preferred_element_type=jnp.float32)
    out_ref[0] = jnp.dot(t.astype(jnp.bfloat16), ab,
                         preferred_element_type=jnp.float32)


def kernel(input) -> jnp.ndarray:
    x = input
    b, nc, hdim, wdim = x.shape
    r = wdim // 40
    bh = _BH
    ch = _CH
    h_blks = hdim // bh

    g = pl.pallas_call(
        functools.partial(_grad_kernel, h_blks=h_blks, ch=ch),
        grid=(b * h_blks, nc // ch),
        in_specs=[
            pl.BlockSpec(
                (1, ch, bh, wdim),
                lambda i, cb: (i // h_blks, cb, i % h_blks, 0),
            ),
            pl.BlockSpec(
                (1, ch, 8, wdim),
                lambda i, cb: (
                    i // h_blks,
                    cb,
                    jnp.maximum((i % h_blks) * (bh // 8) - 1, 0),
                    0,
                ),
            ),
        ],
        out_specs=pl.BlockSpec((1, bh, wdim), lambda i, cb: (i // h_blks, i % h_blks, 0)),
        out_shape=jax.ShapeDtypeStruct((b, hdim, wdim), jnp.bfloat16),
        compiler_params=pltpu.CompilerParams(
            dimension_semantics=("parallel", "arbitrary"),
            vmem_limit_bytes=48 * 1024 * 1024,
        ),
    )(x, x)

    idx = jnp.arange(hdim)
    band = (jnp.abs(idx[:, None] - idx[None, :]) <= r).astype(jnp.bfloat16)

    out = pl.pallas_call(
        _box_kernel,
        grid=(b,),
        in_specs=[
            pl.BlockSpec((1, hdim, wdim), lambda bi: (bi, 0, 0)),
            pl.BlockSpec((hdim, hdim), lambda bi: (0, 0)),
        ],
        out_specs=pl.BlockSpec((1, hdim, wdim), lambda bi: (bi, 0, 0)),
        out_shape=jax.ShapeDtypeStruct((b, hdim, wdim), jnp.float32),
        compiler_params=pltpu.CompilerParams(
            dimension_semantics=("parallel",),
            vmem_limit_bytes=48 * 1024 * 1024,
        ),
    )(g, band)
    return out
